# Initial kernel scaffold; baseline (speedup 1.0000x reference)
import jax
import jax.numpy as jnp
from jax.experimental import pallas as pl


def kernel(x, edge_index):
    raise NotImplementedError("write your pallas kernel here")


# speedup vs baseline: 3.9832x; 3.9832x over previous
import functools

import jax
import jax.numpy as jnp
from jax import lax
from jax.experimental import pallas as pl
from jax.experimental.pallas import tpu as pltpu
from jax.experimental.pallas import tpu_sc as plsc

_NUM_CORES = 2
_NUM_SUBCORES = 16
_NW = _NUM_CORES * _NUM_SUBCORES
_LANES = 16

_CHUNK = 128


def _sqrt16(a):
    ai = lax.bitcast_convert_type(a, jnp.int32)
    yi = jnp.int32(0x5F3759DF) - (ai >> 1)
    y = lax.bitcast_convert_type(yi, jnp.float32)
    for _ in range(3):
        y = y * (1.5 - 0.5 * a * y * y)
    return a * y


def _edge_score_body(x_hbm, row_hbm, col_hbm, ni_hbm,
                     row_v, col_v, a_v, b_v, t_v, ni_v, sem_a, sem_b,
                     *, e_real, e_per_w, n_pad, c_chunks):
    wid = lax.axis_index("s") * _NUM_CORES + lax.axis_index("c")
    base = wid * e_per_w

    zeros16 = jnp.zeros((_LANES,), jnp.float32)

    def _zero(i, carry):
        ni_v[pl.ds(i * _LANES, _LANES)] = zeros16
        return carry

    lax.fori_loop(0, n_pad // _LANES, _zero, 0)

    def _chunk_body(ch, carry):
        off = base + ch * _CHUNK
        pltpu.sync_copy(row_hbm.at[pl.ds(off, _CHUNK)], row_v)
        pltpu.sync_copy(col_hbm.at[pl.ds(off, _CHUNK)], col_v)
        cp_a = pltpu.async_copy(x_hbm.at[row_v], a_v, sem_a)
        cp_b = pltpu.async_copy(x_hbm.at[col_v], b_v, sem_b)
        cp_a.wait()
        cp_b.wait()

        def _group_body(g, gcarry):
            gbase = g * _LANES
            for e in range(_LANES):
                r = gbase + e
                acc = jnp.zeros((_LANES,), jnp.float32)
                for c in range(c_chunks):
                    av = a_v[r, pl.ds(c * _LANES, _LANES)]
                    bv = b_v[r, pl.ds(c * _LANES, _LANES)]
                    d = av - bv
                    acc = acc + d * d
                t_v[pl.ds(e * _LANES, _LANES)] = acc
            lanes = lax.iota(jnp.int32, _LANES)
            colidx = lanes * _LANES
            dsq = jnp.zeros((_LANES,), jnp.float32)
            for c in range(_LANES):
                dsq = dsq + plsc.load_gather(t_v, [colidx + c])
            dist = _sqrt16(dsq + 1e-6)
            gid = off + gbase + lanes
            mask = gid < e_real
            ridx = row_v[pl.ds(gbase, _LANES)]
            plsc.addupdate_scatter(ni_v, [ridx], dist, mask=mask)
            return gcarry

        lax.fori_loop(0, _CHUNK // _LANES, _group_body, 0)
        return carry

    lax.fori_loop(0, e_per_w // _CHUNK, _chunk_body, 0)
    pltpu.sync_copy(ni_v, ni_hbm.at[wid])


def _sc_edge_scores(x2d, rowp, colp, *, e_real, n_pad):
    n, c = x2d.shape
    e_pad = rowp.shape[0]
    e_per_w = e_pad // _NW
    mesh = plsc.VectorSubcoreMesh(
        core_axis_name="c", subcore_axis_name="s",
        num_cores=_NUM_CORES, num_subcores=_NUM_SUBCORES)
    kfn = pl.kernel(
        functools.partial(_edge_score_body, e_real=e_real, e_per_w=e_per_w,
                          n_pad=n_pad, c_chunks=c // _LANES),
        out_type=jax.ShapeDtypeStruct((_NW, n_pad), jnp.float32),
        mesh=mesh,
        compiler_params=pltpu.CompilerParams(needs_layout_passes=False),
        scratch_types=[
            pltpu.VMEM((_CHUNK,), jnp.int32),
            pltpu.VMEM((_CHUNK,), jnp.int32),
            pltpu.VMEM((_CHUNK, c), jnp.float32),
            pltpu.VMEM((_CHUNK, c), jnp.float32),
            pltpu.VMEM((_LANES * _LANES,), jnp.float32),
            pltpu.VMEM((n_pad,), jnp.float32),
            pltpu.SemaphoreType.DMA,
            pltpu.SemaphoreType.DMA,
        ],
    )
    return kfn(x2d, rowp, colp)


def _pool_body(xe_ref, xo_ref, nie_ref, nio_ref, out_ref):
    we = jax.nn.sigmoid(jnp.sum(nie_ref[...], axis=1, keepdims=True))
    wo = jax.nn.sigmoid(jnp.sum(nio_ref[...], axis=1, keepdims=True))
    out_ref[...] = 0.5 * (xe_ref[...] * we + xo_ref[...] * wo)


def _tc_pool(xe, xo, nie_t, nio_t):
    k, c = xe.shape
    blk = 1000
    grid = k // blk
    return pl.pallas_call(
        _pool_body,
        grid=(grid,),
        in_specs=[
            pl.BlockSpec((blk, c), lambda i: (i, 0)),
            pl.BlockSpec((blk, c), lambda i: (i, 0)),
            pl.BlockSpec((blk, _NW), lambda i: (i, 0)),
            pl.BlockSpec((blk, _NW), lambda i: (i, 0)),
        ],
        out_specs=pl.BlockSpec((blk, c), lambda i: (i, 0)),
        out_shape=jax.ShapeDtypeStruct((k, c), jnp.float32),
    )(xe, xo, nie_t, nio_t)


def kernel(x, edge_index):
    b, n, c = x.shape
    e = edge_index.shape[1]
    num_keep = max(1, int(n * 0.5))

    x2d = x.reshape(n, c)
    row = edge_index[0].astype(jnp.int32)
    col = edge_index[1].astype(jnp.int32)

    tile_e = _NW * _CHUNK
    e_pad = ((e + tile_e - 1) // tile_e) * tile_e
    rowp = jnp.zeros((e_pad,), jnp.int32).at[:e].set(row)
    colp = jnp.zeros((e_pad,), jnp.int32).at[:e].set(col)
    n_pad = ((n + _LANES - 1) // _LANES) * _LANES

    nip = _sc_edge_scores(x2d, rowp, colp, e_real=e, n_pad=n_pad)

    ni = nip[:, :n]
    nie_t = ni[:, 0::2].T
    nio_t = ni[:, 1::2].T
    xe = x2d[0::2]
    xo = x2d[1::2]

    pooled = _tc_pool(xe, xo, nie_t, nio_t)
    x_pooled = pooled.reshape(b, num_keep, c)

    idx = jnp.arange(num_keep)
    left = idx[:-1]
    right = idx[1:]
    new_edge_index = jnp.concatenate(
        [jnp.stack([left, right], axis=0), jnp.stack([right, left], axis=0)],
        axis=1)
    return x_pooled, new_edge_index
